# baseline (speedup 1.0000x reference)
import jax
import jax.numpy as jnp
from jax.experimental import pallas as pl
from jax.experimental.pallas import tpu as pltpu
from jax.experimental.pallas import tpu_sc as plsc

RB = 8
LANES = 16


def _sc_add(x, pos):
    n, d = x.shape
    lrows, _ = pos.shape
    n_lblocks = lrows // RB
    n_b = n // lrows

    mesh = plsc.VectorSubcoreMesh(core_axis_name="core",
                                  subcore_axis_name="subcore")

    @pl.kernel(out_type=jax.ShapeDtypeStruct((n, d), x.dtype), mesh=mesh)
    def sc_kernel(x_hbm, pos_hbm, o_hbm):
        def body(x_vmem, pos_vmem, o_vmem):
            @pl.loop(0, RB)
            def _(r):
                @pl.loop(0, d, step=LANES)
                def _(c):
                    slc = (pl.ds(r, 1), pl.ds(c, LANES))
                    o_vmem.at[*slc][...] = (
                        x_vmem.at[*slc][...] + pos_vmem.at[*slc][...]
                    )

        pltpu.emit_pipeline(
            body,
            grid=(n_lblocks, n_b),
            in_specs=[
                pl.BlockSpec((RB, d), index_map=lambda i, j: (j * n_lblocks + i, 0)),
                pl.BlockSpec((RB, d), index_map=lambda i, j: (i, 0)),
            ],
            out_specs=[
                pl.BlockSpec((RB, d), index_map=lambda i, j: (j * n_lblocks + i, 0)),
            ],
            core_axis_name=("core", "subcore"),
            dimension_semantics=(pltpu.PARALLEL, pltpu.ARBITRARY),
        )(x_hbm, pos_hbm, o_hbm)

    return sc_kernel(x, pos)


def kernel(x, pos_emb_weight):
    b, l, d = x.shape
    out = _sc_add(x.reshape(b * l, d), pos_emb_weight)
    return out.reshape(b, l, d)


# speedup vs baseline: 3.9492x; 3.9492x over previous
import jax
import jax.numpy as jnp
from jax.experimental import pallas as pl

BL = 2048


def _add_kernel(x_ref, pos_ref, o_ref):
    o_ref[...] = x_ref[...] + pos_ref[...]


def kernel(x, pos_emb_weight):
    b, l, d = x.shape
    grid = (l // BL, b)
    return pl.pallas_call(
        _add_kernel,
        grid=grid,
        in_specs=[
            pl.BlockSpec((1, BL, d), lambda i, j: (j, i, 0)),
            pl.BlockSpec((BL, d), lambda i, j: (i, 0)),
        ],
        out_specs=pl.BlockSpec((1, BL, d), lambda i, j: (j, i, 0)),
        out_shape=jax.ShapeDtypeStruct((b, l, d), x.dtype),
    )(x, pos_emb_weight)
